# baseline (speedup 1.0000x reference)
import functools

import jax
import jax.numpy as jnp
from jax import lax
from jax.experimental import pallas as pl
from jax.experimental.pallas import tpu as pltpu
from jax.experimental.pallas import tpu_sc as plsc

B = 16384
DIM = 64
NCTX = 5
NC = 2
NS = 16
NW = NC * NS
BPW = B // NW
CH = 64
NCHUNK = BPW // CH
LANES = 16
NG = CH // LANES
CIB = 3
BN = 8192
Q = 253952
NDP = DIM // 2


def _rn_bf16_bits(x):
    b = lax.bitcast_convert_type(x, jnp.int32)
    lsb = lax.shift_right_logical(b, 16) & 1
    return lax.shift_right_logical(b + 0x7FFF + lsb, 16)


def _tc_pack(table):
    t_t = table.T
    eye = jnp.eye(DIM, dtype=jnp.float32)
    sel_even = jnp.zeros((DIM, NDP), jnp.float32).at[
        2 * jnp.arange(NDP), jnp.arange(NDP)].set(1.0)
    sel_odd = jnp.zeros((DIM, NDP), jnp.float32).at[
        2 * jnp.arange(NDP) + 1, jnp.arange(NDP)].set(1.0)

    def body(a0, a1, a2, a3, eye_ref, se_ref, so_ref, o_ref):
        e = eye_ref[...]
        se, so = se_ref[...], so_ref[...]
        dn = (((0,), (0,)), ((), ()))
        dn2 = (((1,), (0,)), ((), ()))
        for qq, a in enumerate((a0, a1, a2, a3)):
            t = lax.dot_general(a[...], e, dn,
                                preferred_element_type=jnp.float32)
            lo = _rn_bf16_bits(lax.dot_general(
                t, se, dn2, preferred_element_type=jnp.float32))
            hi = _rn_bf16_bits(lax.dot_general(
                t, so, dn2, preferred_element_type=jnp.float32))
            o_ref[:, qq * NDP:(qq + 1) * NDP] = lo | lax.shift_left(hi, 16)

    nblk = Q // BN
    vmax = 1000000 // BN

    def in_spec(qq):
        return pl.BlockSpec(
            (DIM, BN),
            lambda i, q=qq: (0, jnp.minimum(q * nblk + i, vmax)))

    return pl.pallas_call(
        body,
        grid=(nblk,),
        in_specs=[in_spec(0), in_spec(1), in_spec(2), in_spec(3),
                  pl.BlockSpec((DIM, DIM), lambda i: (0, 0)),
                  pl.BlockSpec((DIM, NDP), lambda i: (0, 0)),
                  pl.BlockSpec((DIM, NDP), lambda i: (0, 0))],
        out_specs=pl.BlockSpec((BN, 128), lambda i: (i, 0)),
        out_shape=jax.ShapeDtypeStruct((Q, 128), jnp.int32),
    )(t_t, t_t, t_t, t_t, eye, sel_even, sel_odd)


def _quarter(r):
    qq = ((r >= Q).astype(jnp.int32) + (r >= 2 * Q).astype(jnp.int32)
          + (r >= 3 * Q).astype(jnp.int32))
    return qq, r - qq * Q


def _unpack2(w):
    lo = plsc.bitcast(lax.shift_left(w, 16), jnp.float32)
    hi = plsc.bitcast(w & jnp.int32(-65536), jnp.float32)
    return lo, hi


def _make_kernel():
    mesh = plsc.VectorSubcoreMesh(core_axis_name="c", subcore_axis_name="s")

    @functools.partial(
        pl.kernel,
        out_type=jax.ShapeDtypeStruct((B * NCTX,), jnp.float32),
        mesh=mesh,
        compiler_params=pltpu.CompilerParams(needs_layout_passes=False),
        scratch_types=[
            pltpu.VMEM((1, CH), jnp.int32),
            pltpu.VMEM((CIB, 128), jnp.int32),
            pltpu.VMEM((1, CH), jnp.int32),
            pltpu.VMEM((CIB, 128), jnp.int32),
            pltpu.VMEM((CH, 128), jnp.int32),
            pltpu.VMEM((CH * NCTX, 128), jnp.int32),
            pltpu.VMEM((CH * NCTX,), jnp.float32),
            pltpu.SemaphoreType.DMA,
            pltpu.SemaphoreType.DMA,
        ],
    )
    def body(tgt_hbm, ctx_hbm, ttab_hbm, ctab_hbm, out_hbm,
             traw, craw, tidx, cidx, tgt_sl, ctx_sl, out_v, sem, sem2):
        wid = lax.axis_index("s") * NC + lax.axis_index("c")
        lane = lax.iota(jnp.int32, LANES)

        @pl.loop(0, NCHUNK)
        def _chunk(ch):
            base = (wid * NCHUNK + ch) * CH
            cb = base * NCTX
            icps = [pltpu.async_copy(tgt_hbm.at[pl.ds(base, CH)],
                                     traw.at[0], sem2),
                    pltpu.async_copy(ctx_hbm.at[pl.ds(cb, 128)],
                                     craw.at[0], sem2),
                    pltpu.async_copy(ctx_hbm.at[pl.ds(cb + 128, 128)],
                                     craw.at[1], sem2),
                    pltpu.async_copy(ctx_hbm.at[pl.ds(cb + 256, 64)],
                                     craw.at[2, pl.ds(0, 64)], sem2)]
            for cp in icps:
                cp.wait()
            for v in range(CH // LANES):
                _, sl = _quarter(traw[0, pl.ds(v * LANES, LANES)])
                tidx[0, pl.ds(v * LANES, LANES)] = sl
            for j in range(CIB):
                n = 128 if j < 2 else 64
                for v in range(n // LANES):
                    _, sl = _quarter(craw[j, pl.ds(v * LANES, LANES)])
                    cidx[j, pl.ds(v * LANES, LANES)] = sl
            cps = [pltpu.async_copy(ttab_hbm.at[tidx.at[0]], tgt_sl, sem),
                   pltpu.async_copy(ctab_hbm.at[cidx.at[0]],
                                    ctx_sl.at[pl.ds(0, 128)], sem),
                   pltpu.async_copy(ctab_hbm.at[cidx.at[1]],
                                    ctx_sl.at[pl.ds(128, 128)], sem),
                   pltpu.async_copy(ctab_hbm.at[cidx.at[2, pl.ds(0, 64)]],
                                    ctx_sl.at[pl.ds(256, 64)], sem)]
            for cp in cps:
                cp.wait()

            @pl.loop(0, NG)
            def _grp(g):
                trow = g * LANES + lane
                tqq, _ = _quarter(plsc.load_gather(traw.at[0], [trow]))
                tcol = tqq * NDP
                pvecs, ccol, accs = [], [], []
                for c in range(NCTX):
                    p = trow * NCTX + c
                    cqq, _ = _quarter(plsc.load_gather(craw,
                                                       [p >> 7, p & 127]))
                    pvecs.append(p)
                    ccol.append(cqq * NDP)
                    accs.append(jnp.zeros((LANES,), jnp.float32))
                for dp in range(NDP):
                    tlo, thi = _unpack2(
                        plsc.load_gather(tgt_sl, [trow, tcol + dp]))
                    for c in range(NCTX):
                        clo, chi = _unpack2(
                            plsc.load_gather(ctx_sl, [pvecs[c],
                                                      ccol[c] + dp]))
                        accs[c] = accs[c] + clo * tlo + chi * thi
                for c in range(NCTX):
                    plsc.store_scatter(out_v, [pvecs[c]], accs[c])

            pltpu.sync_copy(out_v, out_hbm.at[pl.ds(cb, CH * NCTX)])

    return body


_sc_kernel = _make_kernel()


def kernel(target, context, target_table, context_table):
    tgt1 = target.reshape(B).astype(jnp.int32)
    ctx1 = context.reshape(B * NCTX).astype(jnp.int32)
    ttab = _tc_pack(target_table)
    ctab = _tc_pack(context_table)
    flat = _sc_kernel(tgt1, ctx1, ttab, ctab)
    return flat.reshape(B, NCTX)


# speedup vs baseline: 1.6139x; 1.6139x over previous
import functools

import jax
import jax.numpy as jnp
from jax import lax
from jax.experimental import pallas as pl
from jax.experimental.pallas import tpu as pltpu
from jax.experimental.pallas import tpu_sc as plsc

B = 16384
DIM = 64
NCTX = 5
NC = 2
NS = 16
NW = NC * NS
BPW = B // NW
CH = 64
NCHUNK = BPW // CH
LANES = 16
NG = CH // LANES
CIB = 3
BN = 8192
Q = 253952
NDP = DIM // 2


def _rn_bf16_bits(x):
    b = lax.bitcast_convert_type(x, jnp.int32)
    lsb = lax.shift_right_logical(b, 16) & 1
    return lax.shift_right_logical(b + 0x7FFF + lsb, 16)


def _tc_pack(table):
    t_t = table.T
    sel_even = jnp.zeros((DIM, NDP), jnp.bfloat16).at[
        2 * jnp.arange(NDP), jnp.arange(NDP)].set(1.0)
    sel_odd = jnp.zeros((DIM, NDP), jnp.bfloat16).at[
        2 * jnp.arange(NDP) + 1, jnp.arange(NDP)].set(1.0)

    def body(a0, a1, a2, a3, se_ref, so_ref, o_ref):
        se, so = se_ref[...], so_ref[...]
        dn = (((0,), (0,)), ((), ()))
        for qq, a in enumerate((a0, a1, a2, a3)):
            ab = a[...].astype(jnp.bfloat16)
            lo = lax.bitcast_convert_type(
                lax.dot_general(ab, se, dn,
                                preferred_element_type=jnp.float32),
                jnp.int32)
            hi = lax.bitcast_convert_type(
                lax.dot_general(ab, so, dn,
                                preferred_element_type=jnp.float32),
                jnp.int32)
            o_ref[:, qq * NDP:(qq + 1) * NDP] = (
                lax.shift_right_logical(lo, 16) | (hi & jnp.int32(-65536)))

    nblk = Q // BN
    vmax = 1000000 // BN

    def in_spec(qq):
        return pl.BlockSpec(
            (DIM, BN),
            lambda i, q=qq: (0, jnp.minimum(q * nblk + i, vmax)))

    return pl.pallas_call(
        body,
        grid=(nblk,),
        in_specs=[in_spec(0), in_spec(1), in_spec(2), in_spec(3),
                  pl.BlockSpec((DIM, NDP), lambda i: (0, 0)),
                  pl.BlockSpec((DIM, NDP), lambda i: (0, 0))],
        out_specs=pl.BlockSpec((BN, 128), lambda i: (i, 0)),
        out_shape=jax.ShapeDtypeStruct((Q, 128), jnp.int32),
    )(t_t, t_t, t_t, t_t, sel_even, sel_odd)


def _quarter(r):
    qq = ((r >= Q).astype(jnp.int32) + (r >= 2 * Q).astype(jnp.int32)
          + (r >= 3 * Q).astype(jnp.int32))
    return qq, r - qq * Q


def _unpack2(w):
    lo = plsc.bitcast(lax.shift_left(w, 16), jnp.float32)
    hi = plsc.bitcast(w & jnp.int32(-65536), jnp.float32)
    return lo, hi


def _make_kernel():
    mesh = plsc.VectorSubcoreMesh(core_axis_name="c", subcore_axis_name="s")

    @functools.partial(
        pl.kernel,
        out_type=jax.ShapeDtypeStruct((B * NCTX,), jnp.float32),
        mesh=mesh,
        compiler_params=pltpu.CompilerParams(needs_layout_passes=False),
        scratch_types=[
            pltpu.VMEM((1, CH), jnp.int32),
            pltpu.VMEM((CIB, 128), jnp.int32),
            pltpu.VMEM((1, CH), jnp.int32),
            pltpu.VMEM((CIB, 128), jnp.int32),
            pltpu.VMEM((CH, 128), jnp.int32),
            pltpu.VMEM((CH * NCTX, 128), jnp.int32),
            pltpu.VMEM((CH * NCTX,), jnp.float32),
            pltpu.SemaphoreType.DMA,
            pltpu.SemaphoreType.DMA,
        ],
    )
    def body(tgt_hbm, ctx_hbm, ttab_hbm, ctab_hbm, out_hbm,
             traw, craw, tidx, cidx, tgt_sl, ctx_sl, out_v, sem, sem2):
        wid = lax.axis_index("s") * NC + lax.axis_index("c")
        lane = lax.iota(jnp.int32, LANES)

        @pl.loop(0, NCHUNK)
        def _chunk(ch):
            base = (wid * NCHUNK + ch) * CH
            cb = base * NCTX
            icps = [pltpu.async_copy(tgt_hbm.at[pl.ds(base, CH)],
                                     traw.at[0], sem2),
                    pltpu.async_copy(ctx_hbm.at[pl.ds(cb, 128)],
                                     craw.at[0], sem2),
                    pltpu.async_copy(ctx_hbm.at[pl.ds(cb + 128, 128)],
                                     craw.at[1], sem2),
                    pltpu.async_copy(ctx_hbm.at[pl.ds(cb + 256, 64)],
                                     craw.at[2, pl.ds(0, 64)], sem2)]
            for cp in icps:
                cp.wait()
            for v in range(CH // LANES):
                _, sl = _quarter(traw[0, pl.ds(v * LANES, LANES)])
                tidx[0, pl.ds(v * LANES, LANES)] = sl
            for j in range(CIB):
                n = 128 if j < 2 else 64
                for v in range(n // LANES):
                    _, sl = _quarter(craw[j, pl.ds(v * LANES, LANES)])
                    cidx[j, pl.ds(v * LANES, LANES)] = sl
            cps = [pltpu.async_copy(ttab_hbm.at[tidx.at[0]], tgt_sl, sem),
                   pltpu.async_copy(ctab_hbm.at[cidx.at[0]],
                                    ctx_sl.at[pl.ds(0, 128)], sem),
                   pltpu.async_copy(ctab_hbm.at[cidx.at[1]],
                                    ctx_sl.at[pl.ds(128, 128)], sem),
                   pltpu.async_copy(ctab_hbm.at[cidx.at[2, pl.ds(0, 64)]],
                                    ctx_sl.at[pl.ds(256, 64)], sem)]
            for cp in cps:
                cp.wait()

            @pl.loop(0, NG)
            def _grp(g):
                trow = g * LANES + lane
                tqq, _ = _quarter(plsc.load_gather(traw.at[0], [trow]))
                tcol = tqq * NDP
                pvecs, ccol, accs = [], [], []
                for c in range(NCTX):
                    p = trow * NCTX + c
                    cqq, _ = _quarter(plsc.load_gather(craw,
                                                       [p >> 7, p & 127]))
                    pvecs.append(p)
                    ccol.append(cqq * NDP)
                    accs.append(jnp.zeros((LANES,), jnp.float32))
                for dp in range(NDP):
                    tlo, thi = _unpack2(
                        plsc.load_gather(tgt_sl, [trow, tcol + dp]))
                    for c in range(NCTX):
                        clo, chi = _unpack2(
                            plsc.load_gather(ctx_sl, [pvecs[c],
                                                      ccol[c] + dp]))
                        accs[c] = accs[c] + clo * tlo + chi * thi
                for c in range(NCTX):
                    plsc.store_scatter(out_v, [pvecs[c]], accs[c])

            pltpu.sync_copy(out_v, out_hbm.at[pl.ds(cb, CH * NCTX)])

    return body


_sc_kernel = _make_kernel()


def kernel(target, context, target_table, context_table):
    tgt1 = target.reshape(B).astype(jnp.int32)
    ctx1 = context.reshape(B * NCTX).astype(jnp.int32)
    ttab = _tc_pack(target_table)
    ctab = _tc_pack(context_table)
    flat = _sc_kernel(tgt1, ctx1, ttab, ctab)
    return flat.reshape(B, NCTX)


# speedup vs baseline: 1.7187x; 1.0650x over previous
import functools

import jax
import jax.numpy as jnp
from jax import lax
from jax.experimental import pallas as pl
from jax.experimental.pallas import tpu as pltpu
from jax.experimental.pallas import tpu_sc as plsc

B = 16384
DIM = 64
NCTX = 5
NC = 2
NS = 16
NW = NC * NS
BPW = B // NW
CH = 128
NCHUNK = BPW // CH
LANES = 16
NG = CH // LANES
BN = 8192
Q = 253952
NDP = DIM // 2


def _tc_pack(table):
    t_t = table.T
    sel_even = jnp.zeros((DIM, NDP), jnp.bfloat16).at[
        2 * jnp.arange(NDP), jnp.arange(NDP)].set(1.0)
    sel_odd = jnp.zeros((DIM, NDP), jnp.bfloat16).at[
        2 * jnp.arange(NDP) + 1, jnp.arange(NDP)].set(1.0)

    def body(a0, a1, a2, a3, se_ref, so_ref, o_ref):
        se, so = se_ref[...], so_ref[...]
        dn = (((0,), (0,)), ((), ()))
        for qq, a in enumerate((a0, a1, a2, a3)):
            ab = a[...].astype(jnp.bfloat16)
            lo = lax.bitcast_convert_type(
                lax.dot_general(ab, se, dn,
                                preferred_element_type=jnp.float32),
                jnp.int32)
            hi = lax.bitcast_convert_type(
                lax.dot_general(ab, so, dn,
                                preferred_element_type=jnp.float32),
                jnp.int32)
            o_ref[:, qq * NDP:(qq + 1) * NDP] = (
                lax.shift_right_logical(lo, 16) | (hi & jnp.int32(-65536)))

    nblk = Q // BN
    vmax = 1000000 // BN

    def in_spec(qq):
        return pl.BlockSpec(
            (DIM, BN),
            lambda i, q=qq: (0, jnp.minimum(q * nblk + i, vmax)))

    return pl.pallas_call(
        body,
        grid=(nblk,),
        in_specs=[in_spec(0), in_spec(1), in_spec(2), in_spec(3),
                  pl.BlockSpec((DIM, NDP), lambda i: (0, 0)),
                  pl.BlockSpec((DIM, NDP), lambda i: (0, 0))],
        out_specs=pl.BlockSpec((BN, 128), lambda i: (i, 0)),
        out_shape=jax.ShapeDtypeStruct((Q, 128), jnp.int32),
    )(t_t, t_t, t_t, t_t, sel_even, sel_odd)


def _quarter(r):
    qq = ((r >= Q).astype(jnp.int32) + (r >= 2 * Q).astype(jnp.int32)
          + (r >= 3 * Q).astype(jnp.int32))
    return qq, r - qq * Q


def _unpack2(w):
    lo = plsc.bitcast(lax.shift_left(w, 16), jnp.float32)
    hi = plsc.bitcast(w & jnp.int32(-65536), jnp.float32)
    return lo, hi


def _make_kernel():
    mesh = plsc.VectorSubcoreMesh(core_axis_name="c", subcore_axis_name="s")

    @functools.partial(
        pl.kernel,
        out_type=jax.ShapeDtypeStruct((B * NCTX,), jnp.float32),
        mesh=mesh,
        compiler_params=pltpu.CompilerParams(needs_layout_passes=False),
        scratch_types=[
            pltpu.VMEM((1, CH), jnp.int32),
            pltpu.VMEM((NCTX, CH), jnp.int32),
            pltpu.VMEM((1, CH), jnp.int32),
            pltpu.VMEM((NCTX, CH), jnp.int32),
            pltpu.VMEM((CH, 128), jnp.int32),
            pltpu.VMEM((CH * NCTX, 128), jnp.int32),
            pltpu.VMEM((CH * NCTX,), jnp.float32),
            pltpu.SemaphoreType.DMA,
            pltpu.SemaphoreType.DMA,
        ],
    )
    def body(tgt_hbm, ctx_hbm, ttab_hbm, ctab_hbm, out_hbm,
             traw, craw, tidx, cidx, tgt_sl, ctx_sl, out_v, sem, sem2):
        wid = lax.axis_index("s") * NC + lax.axis_index("c")
        lane = lax.iota(jnp.int32, LANES)

        @pl.loop(0, NCHUNK)
        def _chunk(ch):
            base = (wid * NCHUNK + ch) * CH
            icps = [pltpu.async_copy(tgt_hbm.at[pl.ds(base, CH)],
                                     traw.at[0], sem2)]
            icps += [pltpu.async_copy(ctx_hbm.at[pl.ds(c * B + base, CH)],
                                      craw.at[c], sem2) for c in range(NCTX)]
            for cp in icps:
                cp.wait()
            for v in range(CH // LANES):
                _, sl = _quarter(traw[0, pl.ds(v * LANES, LANES)])
                tidx[0, pl.ds(v * LANES, LANES)] = sl
            for c in range(NCTX):
                for v in range(CH // LANES):
                    _, sl = _quarter(craw[c, pl.ds(v * LANES, LANES)])
                    cidx[c, pl.ds(v * LANES, LANES)] = sl
            cps = [pltpu.async_copy(ttab_hbm.at[tidx.at[0]], tgt_sl, sem)]
            cps += [pltpu.async_copy(ctab_hbm.at[cidx.at[c]],
                                     ctx_sl.at[pl.ds(c * CH, CH)], sem)
                    for c in range(NCTX)]
            for cp in cps:
                cp.wait()

            @pl.loop(0, NG)
            def _grp(g):
                trow = g * LANES + lane
                tqq, _ = _quarter(plsc.load_gather(traw.at[0], [trow]))
                tcol = tqq * NDP
                pvecs, ccol, accs = [], [], []
                for c in range(NCTX):
                    p = c * CH + trow
                    cqq, _ = _quarter(plsc.load_gather(craw.at[c], [trow]))
                    pvecs.append(p)
                    ccol.append(cqq * NDP)
                    accs.append(jnp.zeros((LANES,), jnp.float32))
                for dp in range(NDP):
                    tlo, thi = _unpack2(
                        plsc.load_gather(tgt_sl, [trow, tcol + dp]))
                    for c in range(NCTX):
                        clo, chi = _unpack2(
                            plsc.load_gather(ctx_sl, [pvecs[c],
                                                      ccol[c] + dp]))
                        accs[c] = accs[c] + clo * tlo + chi * thi
                for c in range(NCTX):
                    plsc.store_scatter(out_v, [pvecs[c]], accs[c])

            for c in range(NCTX):
                pltpu.sync_copy(out_v.at[pl.ds(c * CH, CH)],
                                out_hbm.at[pl.ds(c * B + base, CH)])

    return body


_sc_kernel = _make_kernel()


def kernel(target, context, target_table, context_table):
    tgt1 = target.reshape(B).astype(jnp.int32)
    ctx1 = context.T.reshape(B * NCTX).astype(jnp.int32)
    ttab = _tc_pack(target_table)
    ctab = _tc_pack(context_table)
    flat = _sc_kernel(tgt1, ctx1, ttab, ctab)
    return flat.reshape(NCTX, B).T
